# baseline (speedup 1.0000x reference)
import functools

import jax
import jax.numpy as jnp
from jax import lax
from jax.experimental import pallas as pl
from jax.experimental.pallas import tpu as pltpu
from jax.experimental.pallas import tpu_sc as plsc

N = 10000
E = 320000
D_IN = 128
D_H = 64
D_OUT = 64
D_EDGE = 4

NC = 2
NS = 16
NW = NC * NS
L = 16
EPT = E // NW
K = 80
CH = EPT // K
G = K // L
NP = 10240
ROWS = NP // NS
SW = 80
NBLK = 1000
EBLK = 32000
EPS = 1e-16
SLOPE = 0.2


def _node_body(x_ref, w_ref, h_ref):
    h_ref[...] = jnp.dot(x_ref[...], w_ref[...],
                         preferred_element_type=jnp.float32)


def _node_call(x, W):
    d_in = x.shape[1]
    return pl.pallas_call(
        _node_body,
        grid=(N // NBLK,),
        in_specs=[
            pl.BlockSpec((NBLK, d_in), lambda i: (i, 0)),
            pl.BlockSpec((d_in, D_H), lambda i: (0, 0)),
        ],
        out_specs=pl.BlockSpec((NBLK, D_H), lambda i: (i, 0)),
        out_shape=jax.ShapeDtypeStruct((N, D_H), jnp.float32),
    )(x, W)


def _attn_body(h_ref, a_ref, out_ref):
    out_ref[...] = lax.dot_general(
        a_ref[...], h_ref[...], (((1,), (1,)), ((), ())),
        preferred_element_type=jnp.float32)


def _attn_call(h, A8):
    return pl.pallas_call(
        _attn_body,
        in_specs=[
            pl.BlockSpec((N, D_H), lambda: (0, 0)),
            pl.BlockSpec((8, D_H), lambda: (0, 0)),
        ],
        out_specs=pl.BlockSpec((8, N), lambda: (0, 0)),
        out_shape=jax.ShapeDtypeStruct((8, N), jnp.float32),
    )(h, A8)


def _edgevec_body(ea_ref, ws_ref, out_ref):
    out_ref[...] = jnp.dot(ws_ref[...], ea_ref[...],
                           preferred_element_type=jnp.float32)


def _edgevec_call(eaT, Wstack):
    return pl.pallas_call(
        _edgevec_body,
        grid=(E // EBLK,),
        in_specs=[
            pl.BlockSpec((D_EDGE, EBLK), lambda i: (0, i)),
            pl.BlockSpec((8, D_EDGE), lambda i: (0, 0)),
        ],
        out_specs=pl.BlockSpec((8, EBLK), lambda i: (0, i)),
        out_shape=jax.ShapeDtypeStruct((8, E), jnp.float32),
    )(eaT, Wstack)


def _combine(n0_ref, n1_ref, d0_ref, d1_ref, b_ref):
    num = n0_ref[0] + n1_ref[0]
    den = d0_ref[0][:, 0:1] + d1_ref[0][:, 0:1]
    return num / (den + EPS) + b_ref[...]


def _mid_body(n0_ref, n1_ref, d0_ref, d1_ref, b_ref, w_ref, h_ref):
    h1 = jnp.maximum(_combine(n0_ref, n1_ref, d0_ref, d1_ref, b_ref), 0.0)
    h_ref[...] = jnp.dot(h1, w_ref[...], preferred_element_type=jnp.float32)


def _mid_call(num, den, b, W):
    return pl.pallas_call(
        _mid_body,
        grid=(N // NBLK,),
        in_specs=[
            pl.BlockSpec((1, NBLK, D_H), lambda i: (0, i, 0)),
            pl.BlockSpec((1, NBLK, D_H), lambda i: (1, i, 0)),
            pl.BlockSpec((1, NBLK, 16), lambda i: (0, i, 0)),
            pl.BlockSpec((1, NBLK, 16), lambda i: (1, i, 0)),
            pl.BlockSpec((1, D_H), lambda i: (0, 0)),
            pl.BlockSpec((D_H, D_H), lambda i: (0, 0)),
        ],
        out_specs=pl.BlockSpec((NBLK, D_H), lambda i: (i, 0)),
        out_shape=jax.ShapeDtypeStruct((N, D_H), jnp.float32),
    )(num, num, den, den, b, W)


def _final_body(n0_ref, n1_ref, d0_ref, d1_ref, b_ref, wl_ref, bl_ref,
                out_ref):
    h = _combine(n0_ref, n1_ref, d0_ref, d1_ref, b_ref)
    out_ref[...] = jnp.dot(h, wl_ref[...],
                           preferred_element_type=jnp.float32) + bl_ref[...]


def _final_call(num, den, b, Wl, bl):
    return pl.pallas_call(
        _final_body,
        grid=(N // NBLK,),
        in_specs=[
            pl.BlockSpec((1, NBLK, D_H), lambda i: (0, i, 0)),
            pl.BlockSpec((1, NBLK, D_H), lambda i: (1, i, 0)),
            pl.BlockSpec((1, NBLK, 16), lambda i: (0, i, 0)),
            pl.BlockSpec((1, NBLK, 16), lambda i: (1, i, 0)),
            pl.BlockSpec((1, D_H), lambda i: (0, 0)),
            pl.BlockSpec((D_H, D_OUT), lambda i: (0, 0)),
            pl.BlockSpec((1, D_OUT), lambda i: (0, 0)),
        ],
        out_specs=pl.BlockSpec((NBLK, D_OUT), lambda i: (i, 0)),
        out_shape=jax.ShapeDtypeStruct((N, D_OUT), jnp.float32),
    )(num, num, den, den, b, Wl, bl)


def _edge_sc_body(h_hbm, asad_hbm, ei_hbm, ae_hbm, flag_hbm, zero_hbm,
                  num_hbm, den_hbm,
                  as_v, ad_v, src_v, dst_v, ae_v, flag_v, gbuf, sbuf, s_sh,
                  gsem, ssem):
    cid = lax.axis_index("c")
    sid = lax.axis_index("s")
    wid = cid * NS + sid
    row0 = sid * ROWS

    pltpu.sync_copy(flag_hbm, flag_v)
    pltpu.sync_copy(asad_hbm.at[0], as_v)
    pltpu.sync_copy(asad_hbm.at[1], ad_v)
    pltpu.sync_copy(ei_hbm.at[0, wid], src_v)
    pltpu.sync_copy(ei_hbm.at[1, wid], dst_v)
    lidx = flag_v[...][0]
    pltpu.sync_copy(ae_hbm.at[lidx, wid], ae_v)
    pltpu.sync_copy(zero_hbm.at[pl.ds(row0, ROWS)],
                    s_sh.at[pl.ds(row0, ROWS)])
    plsc.subcore_barrier()

    def start_gather(j, b):
        pltpu.async_copy(h_hbm.at[src_v.at[j]], gbuf.at[b],
                         gsem.at[b])

    def wait_gather(j, b):
        pltpu.make_async_copy(h_hbm.at[src_v.at[j]],
                              gbuf.at[b], gsem.at[b]).wait()

    def start_scatter(j, b):
        pltpu.async_copy(sbuf.at[b], s_sh.at[dst_v.at[j]], ssem.at[b],
                         add=True)

    def wait_scatter(j, b):
        pltpu.make_async_copy(sbuf.at[b], s_sh.at[dst_v.at[j]],
                              ssem.at[b]).wait()

    start_gather(0, 0)
    start_gather(1, 1)

    def slot(j, bg, bs):
        wait_gather(j, bg)

        @pl.when(j + 2 < CH)
        def _():
            start_gather(j + 2, (bg + 2) % 3)

        @pl.when(j >= 2)
        def _():
            wait_scatter(j - 2, bs)

        for g in range(G):
            si = src_v[j, pl.ds(g * L, L)]
            di = dst_v[j, pl.ds(g * L, L)]
            aev = ae_v[j, pl.ds(g * L, L)]
            t = (plsc.load_gather(as_v, [si])
                 + plsc.load_gather(ad_v, [di]) + aev)
            t = jnp.maximum(t, SLOPE * t)
            w = jnp.exp(t)
            for r in range(L):
                row = g * L + r
                wr = jnp.full((L,), w[r], jnp.float32)
                sbuf[bs, row, pl.ds(4 * L, L)] = wr
                for c in range(4):
                    sbuf[bs, row, pl.ds(c * L, L)] = (
                        gbuf[bg, row, pl.ds(c * L, L)] * wr)

        start_scatter(j, bs)

    def loop_body(i, carry):
        j0 = 6 * i
        for k in range(6):
            if k == 0:
                slot(j0, 0, 0)
            else:
                @pl.when(j0 + k < CH)
                def _(k=k):
                    slot(j0 + k, k % 3, k % 2)

        return carry

    lax.fori_loop(0, (CH + 5) // 6, loop_body, 0)
    wait_scatter(CH - 2, (CH - 2) % 2)
    wait_scatter(CH - 1, (CH - 1) % 2)
    plsc.subcore_barrier()
    for i in range(ROWS // K):
        r0 = row0 + i * K
        pltpu.sync_copy(s_sh.at[pl.ds(r0, K)], sbuf.at[0])
        pltpu.sync_copy(sbuf.at[0, :, pl.ds(0, D_H)],
                        num_hbm.at[cid, pl.ds(r0, K)])
        pltpu.sync_copy(sbuf.at[0, :, pl.ds(D_H, 16)],
                        den_hbm.at[cid, pl.ds(r0, K)])


_edge_sc_call = functools.partial(
    pl.kernel,
    out_type=[
        jax.ShapeDtypeStruct((NC, NP, D_H), jnp.float32),
        jax.ShapeDtypeStruct((NC, NP, 16), jnp.float32),
    ],
    mesh=plsc.VectorSubcoreMesh(core_axis_name="c", subcore_axis_name="s"),
    compiler_params=pltpu.CompilerParams(needs_layout_passes=False,
                                         use_tc_tiling_on_sc=False),
    scratch_types=[
        pltpu.VMEM((N,), jnp.float32),
        pltpu.VMEM((N,), jnp.float32),
        pltpu.VMEM((CH, K), jnp.int32),
        pltpu.VMEM((CH, K), jnp.int32),
        pltpu.VMEM((CH, K), jnp.float32),
        pltpu.VMEM((L,), jnp.int32),
        pltpu.VMEM((3, K, D_H), jnp.float32),
        pltpu.VMEM((2, K, SW), jnp.float32),
        pltpu.VMEM_SHARED((NP, SW), jnp.float32),
        pltpu.SemaphoreType.DMA((3,)),
        pltpu.SemaphoreType.DMA((2,)),
    ],
)(_edge_sc_body)


def kernel(x, edge_index, edge_attr, W1, a_src1, a_dst1, We1, ae1, b1,
           W2, a_src2, a_dst2, We2, ae2, b2, Wl, bl):
    f32 = jnp.float32
    Wstack = (jnp.zeros((8, D_EDGE), f32)
              .at[0, :].set(We1 @ ae1)
              .at[1, :].set(We2 @ ae2))

    aev4 = _edgevec_call(edge_attr.T, Wstack).reshape(8, NW, CH, K)
    ei4 = edge_index.reshape(2, NW, CH, K)
    zeros = jnp.zeros((NP, SW), f32)
    flag0 = jnp.zeros((L,), jnp.int32)
    flag1 = jnp.ones((L,), jnp.int32)
    A1 = jnp.zeros((8, D_H), f32).at[0].set(a_src1).at[1].set(a_dst1)
    A2 = jnp.zeros((8, D_H), f32).at[0].set(a_src2).at[1].set(a_dst2)

    h1 = _node_call(x, W1)
    asad1 = _attn_call(h1, A1)
    num1, den1 = _edge_sc_call(h1, asad1, ei4, aev4, flag0, zeros)
    h2 = _mid_call(num1, den1, b1.reshape(1, D_H), W2)
    asad2 = _attn_call(h2, A2)
    num2, den2 = _edge_sc_call(h2, asad2, ei4, aev4, flag1, zeros)
    return _final_call(num2, den2, b2.reshape(1, D_H), Wl,
                       bl.reshape(1, D_OUT))


# speedup vs baseline: 1.1807x; 1.1807x over previous
import functools

import jax
import jax.numpy as jnp
from jax import lax
from jax.experimental import pallas as pl
from jax.experimental.pallas import tpu as pltpu
from jax.experimental.pallas import tpu_sc as plsc

N = 10000
E = 320000
D_IN = 128
D_H = 64
D_OUT = 64
D_EDGE = 4

NC = 2
NS = 16
NW = NC * NS
L = 16
EPT = E // NW
K = 80
CH = EPT // K
G = K // L
NP = 10240
ROWS = NP // NS
SW = 80
NBLK = 2000
EBLK = 64000
EPS = 1e-16
SLOPE = 0.2


def _node_body(x_ref, w_ref, h_ref):
    h_ref[...] = jnp.dot(x_ref[...], w_ref[...],
                         preferred_element_type=jnp.float32)


def _node_call(x, W):
    d_in = x.shape[1]
    return pl.pallas_call(
        _node_body,
        grid=(N // NBLK,),
        in_specs=[
            pl.BlockSpec((NBLK, d_in), lambda i: (i, 0)),
            pl.BlockSpec((d_in, D_H), lambda i: (0, 0)),
        ],
        out_specs=pl.BlockSpec((NBLK, D_H), lambda i: (i, 0)),
        out_shape=jax.ShapeDtypeStruct((N, D_H), jnp.float32),
    )(x, W)


def _attn_body(h_ref, a_ref, out_ref):
    out_ref[...] = lax.dot_general(
        a_ref[...], h_ref[...], (((1,), (1,)), ((), ())),
        preferred_element_type=jnp.float32)


def _attn_call(h, A8):
    return pl.pallas_call(
        _attn_body,
        in_specs=[
            pl.BlockSpec((N, D_H), lambda: (0, 0)),
            pl.BlockSpec((8, D_H), lambda: (0, 0)),
        ],
        out_specs=pl.BlockSpec((8, N), lambda: (0, 0)),
        out_shape=jax.ShapeDtypeStruct((8, N), jnp.float32),
    )(h, A8)


def _edgevec_body(ea_ref, ws_ref, out_ref):
    out_ref[...] = jnp.dot(ws_ref[...], ea_ref[...],
                           preferred_element_type=jnp.float32)


def _edgevec_call(eaT, Wstack):
    return pl.pallas_call(
        _edgevec_body,
        grid=(E // EBLK,),
        in_specs=[
            pl.BlockSpec((D_EDGE, EBLK), lambda i: (0, i)),
            pl.BlockSpec((8, D_EDGE), lambda i: (0, 0)),
        ],
        out_specs=pl.BlockSpec((8, EBLK), lambda i: (0, i)),
        out_shape=jax.ShapeDtypeStruct((8, E), jnp.float32),
    )(eaT, Wstack)


def _combine(n0_ref, n1_ref, d0_ref, d1_ref, b_ref):
    num = n0_ref[0] + n1_ref[0]
    den = d0_ref[0][:, 0:1] + d1_ref[0][:, 0:1]
    return num / (den + EPS) + b_ref[...]


def _mid_body(n0_ref, n1_ref, d0_ref, d1_ref, b_ref, w_ref, h_ref):
    h1 = jnp.maximum(_combine(n0_ref, n1_ref, d0_ref, d1_ref, b_ref), 0.0)
    h_ref[...] = jnp.dot(h1, w_ref[...], preferred_element_type=jnp.float32)


def _mid_call(num, den, b, W):
    return pl.pallas_call(
        _mid_body,
        grid=(N // NBLK,),
        in_specs=[
            pl.BlockSpec((1, NBLK, D_H), lambda i: (0, i, 0)),
            pl.BlockSpec((1, NBLK, D_H), lambda i: (1, i, 0)),
            pl.BlockSpec((1, NBLK, 16), lambda i: (0, i, 0)),
            pl.BlockSpec((1, NBLK, 16), lambda i: (1, i, 0)),
            pl.BlockSpec((1, D_H), lambda i: (0, 0)),
            pl.BlockSpec((D_H, D_H), lambda i: (0, 0)),
        ],
        out_specs=pl.BlockSpec((NBLK, D_H), lambda i: (i, 0)),
        out_shape=jax.ShapeDtypeStruct((N, D_H), jnp.float32),
    )(num, num, den, den, b, W)


def _final_body(n0_ref, n1_ref, d0_ref, d1_ref, b_ref, wl_ref, bl_ref,
                out_ref):
    h = _combine(n0_ref, n1_ref, d0_ref, d1_ref, b_ref)
    out_ref[...] = jnp.dot(h, wl_ref[...],
                           preferred_element_type=jnp.float32) + bl_ref[...]


def _final_call(num, den, b, Wl, bl):
    return pl.pallas_call(
        _final_body,
        grid=(N // NBLK,),
        in_specs=[
            pl.BlockSpec((1, NBLK, D_H), lambda i: (0, i, 0)),
            pl.BlockSpec((1, NBLK, D_H), lambda i: (1, i, 0)),
            pl.BlockSpec((1, NBLK, 16), lambda i: (0, i, 0)),
            pl.BlockSpec((1, NBLK, 16), lambda i: (1, i, 0)),
            pl.BlockSpec((1, D_H), lambda i: (0, 0)),
            pl.BlockSpec((D_H, D_OUT), lambda i: (0, 0)),
            pl.BlockSpec((1, D_OUT), lambda i: (0, 0)),
        ],
        out_specs=pl.BlockSpec((NBLK, D_OUT), lambda i: (i, 0)),
        out_shape=jax.ShapeDtypeStruct((N, D_OUT), jnp.float32),
    )(num, num, den, den, b, Wl, bl)


def _edge_sc_body(h_hbm, asad_hbm, ei_hbm, ae_hbm, flag_hbm, zero_hbm,
                  num_hbm, den_hbm,
                  as_v, ad_v, src_v, dst_v, ae_v, flag_v, gbuf, sbuf, s_sh,
                  gsem, ssem):
    cid = lax.axis_index("c")
    sid = lax.axis_index("s")
    wid = cid * NS + sid
    row0 = sid * ROWS

    pltpu.sync_copy(flag_hbm, flag_v)
    pltpu.sync_copy(asad_hbm.at[0], as_v)
    pltpu.sync_copy(asad_hbm.at[1], ad_v)
    pltpu.sync_copy(ei_hbm.at[0, wid], src_v)
    pltpu.sync_copy(ei_hbm.at[1, wid], dst_v)
    lidx = flag_v[...][0]
    pltpu.sync_copy(ae_hbm.at[lidx, wid], ae_v)
    pltpu.sync_copy(zero_hbm.at[pl.ds(row0, ROWS)],
                    s_sh.at[pl.ds(row0, ROWS)])
    plsc.subcore_barrier()

    def start_gather(j, b):
        pltpu.async_copy(h_hbm.at[src_v.at[j]], gbuf.at[b],
                         gsem.at[b])

    def wait_gather(j, b):
        pltpu.make_async_copy(h_hbm.at[src_v.at[j]],
                              gbuf.at[b], gsem.at[b]).wait()

    def start_scatter(j, b):
        pltpu.async_copy(sbuf.at[b], s_sh.at[dst_v.at[j]], ssem.at[b],
                         add=True)

    def wait_scatter(j, b):
        pltpu.make_async_copy(sbuf.at[b], s_sh.at[dst_v.at[j]],
                              ssem.at[b]).wait()

    start_gather(0, 0)
    start_gather(1, 1)
    zi = jnp.zeros((L,), jnp.int32)

    def slot(j, b):
        wait_gather(j, b)

        @pl.when(j >= 2)
        def _():
            wait_scatter(j - 2, b)

        for g in range(G):
            si = src_v[j, pl.ds(g * L, L)]
            di = dst_v[j, pl.ds(g * L, L)]
            aev = ae_v[j, pl.ds(g * L, L)]
            t = (plsc.load_gather(as_v, [si])
                 + plsc.load_gather(ad_v, [di]) + aev)
            t = jnp.maximum(t, SLOPE * t)
            w = jnp.exp(t)
            for r in range(L):
                row = g * L + r
                wr = jnp.full((L,), w[r], jnp.float32)
                sbuf[b, row, pl.ds(4 * L, L)] = wr
                for c in range(4):
                    sbuf[b, row, pl.ds(c * L, L)] = (
                        gbuf[b, row, pl.ds(c * L, L)] * wr)

        @pl.when(j + 2 < CH)
        def _():
            start_gather(j + 2, b)

        start_scatter(j, b)

    def loop_body(i, carry):
        j0 = 2 * i
        slot(j0, 0)

        @pl.when(j0 + 1 < CH)
        def _():
            slot(j0 + 1, 1)

        return carry

    lax.fori_loop(0, (CH + 1) // 2, loop_body, 0)
    wait_scatter(CH - 2, (CH - 2) % 2)
    wait_scatter(CH - 1, (CH - 1) % 2)
    plsc.subcore_barrier()
    for i in range(ROWS // K):
        r0 = row0 + i * K
        pltpu.sync_copy(s_sh.at[pl.ds(r0, K)], sbuf.at[0])
        pltpu.sync_copy(sbuf.at[0, :, pl.ds(0, D_H)],
                        num_hbm.at[cid, pl.ds(r0, K)])
        pltpu.sync_copy(sbuf.at[0, :, pl.ds(D_H, 16)],
                        den_hbm.at[cid, pl.ds(r0, K)])


_edge_sc_call = functools.partial(
    pl.kernel,
    out_type=[
        jax.ShapeDtypeStruct((NC, NP, D_H), jnp.float32),
        jax.ShapeDtypeStruct((NC, NP, 16), jnp.float32),
    ],
    mesh=plsc.VectorSubcoreMesh(core_axis_name="c", subcore_axis_name="s"),
    compiler_params=pltpu.CompilerParams(needs_layout_passes=False,
                                         use_tc_tiling_on_sc=False),
    scratch_types=[
        pltpu.VMEM((N,), jnp.float32),
        pltpu.VMEM((N,), jnp.float32),
        pltpu.VMEM((CH, K), jnp.int32),
        pltpu.VMEM((CH, K), jnp.int32),
        pltpu.VMEM((CH, K), jnp.float32),
        pltpu.VMEM((L,), jnp.int32),
        pltpu.VMEM((2, K, D_H), jnp.float32),
        pltpu.VMEM((2, K, SW), jnp.float32),
        pltpu.VMEM_SHARED((NP, SW), jnp.float32),
        pltpu.SemaphoreType.DMA((2,)),
        pltpu.SemaphoreType.DMA((2,)),
    ],
)(_edge_sc_body)


def kernel(x, edge_index, edge_attr, W1, a_src1, a_dst1, We1, ae1, b1,
           W2, a_src2, a_dst2, We2, ae2, b2, Wl, bl):
    f32 = jnp.float32
    Wstack = (jnp.zeros((8, D_EDGE), f32)
              .at[0, :].set(We1 @ ae1)
              .at[1, :].set(We2 @ ae2))

    aev4 = _edgevec_call(edge_attr.T, Wstack).reshape(8, NW, CH, K)
    ei4 = edge_index.reshape(2, NW, CH, K)
    zeros = jnp.zeros((NP, SW), f32)
    flag0 = jnp.zeros((L,), jnp.int32)
    flag1 = jnp.ones((L,), jnp.int32)
    A1 = jnp.zeros((8, D_H), f32).at[0].set(a_src1).at[1].set(a_dst1)
    A2 = jnp.zeros((8, D_H), f32).at[0].set(a_src2).at[1].set(a_dst2)

    h1 = _node_call(x, W1)
    asad1 = _attn_call(h1, A1)
    num1, den1 = _edge_sc_call(h1, asad1, ei4, aev4, flag0, zeros)
    h2 = _mid_call(num1, den1, b1.reshape(1, D_H), W2)
    asad2 = _attn_call(h2, A2)
    num2, den2 = _edge_sc_call(h2, asad2, ei4, aev4, flag1, zeros)
    return _final_call(num2, den2, b2.reshape(1, D_H), Wl,
                       bl.reshape(1, D_OUT))


# speedup vs baseline: 1.1930x; 1.0104x over previous
import functools

import jax
import jax.numpy as jnp
from jax import lax
from jax.experimental import pallas as pl
from jax.experimental.pallas import tpu as pltpu
from jax.experimental.pallas import tpu_sc as plsc

N = 10000
E = 320000
D_IN = 128
D_H = 64
D_OUT = 64
D_EDGE = 4

NC = 2
NS = 16
NW = NC * NS
L = 16
EPT = E // NW
K = 80
CH = EPT // K
G = K // L
NP = 10240
ROWS = NP // NS
SW = 80
NBLK = 5000
EBLK = 320000
EPS = 1e-16
SLOPE = 0.2


def _node_body(x_ref, w_ref, h_ref):
    h_ref[...] = jnp.dot(x_ref[...], w_ref[...],
                         preferred_element_type=jnp.float32)


def _node_call(x, W):
    d_in = x.shape[1]
    return pl.pallas_call(
        _node_body,
        grid=(N // NBLK,),
        in_specs=[
            pl.BlockSpec((NBLK, d_in), lambda i: (i, 0)),
            pl.BlockSpec((d_in, D_H), lambda i: (0, 0)),
        ],
        out_specs=pl.BlockSpec((NBLK, D_H), lambda i: (i, 0)),
        out_shape=jax.ShapeDtypeStruct((N, D_H), jnp.float32),
    )(x, W)


def _attn_body(h_ref, a_ref, out_ref):
    out_ref[...] = lax.dot_general(
        a_ref[...], h_ref[...], (((1,), (1,)), ((), ())),
        preferred_element_type=jnp.float32)


def _attn_call(h, A8):
    return pl.pallas_call(
        _attn_body,
        in_specs=[
            pl.BlockSpec((N, D_H), lambda: (0, 0)),
            pl.BlockSpec((8, D_H), lambda: (0, 0)),
        ],
        out_specs=pl.BlockSpec((8, N), lambda: (0, 0)),
        out_shape=jax.ShapeDtypeStruct((8, N), jnp.float32),
    )(h, A8)


def _edgevec_body(ea_ref, ws_ref, out_ref):
    out_ref[...] = jnp.dot(ws_ref[...], ea_ref[...],
                           preferred_element_type=jnp.float32)


def _edgevec_call(eaT, Wstack):
    return pl.pallas_call(
        _edgevec_body,
        grid=(E // EBLK,),
        in_specs=[
            pl.BlockSpec((D_EDGE, EBLK), lambda i: (0, i)),
            pl.BlockSpec((8, D_EDGE), lambda i: (0, 0)),
        ],
        out_specs=pl.BlockSpec((8, EBLK), lambda i: (0, i)),
        out_shape=jax.ShapeDtypeStruct((8, E), jnp.float32),
    )(eaT, Wstack)


def _combine(n0_ref, n1_ref, d0_ref, d1_ref, b_ref):
    num = n0_ref[0] + n1_ref[0]
    den = d0_ref[0][:, 0:1] + d1_ref[0][:, 0:1]
    return num / (den + EPS) + b_ref[...]


def _mid_body(n0_ref, n1_ref, d0_ref, d1_ref, b_ref, w_ref, h_ref):
    h1 = jnp.maximum(_combine(n0_ref, n1_ref, d0_ref, d1_ref, b_ref), 0.0)
    h_ref[...] = jnp.dot(h1, w_ref[...], preferred_element_type=jnp.float32)


def _mid_call(num, den, b, W):
    return pl.pallas_call(
        _mid_body,
        grid=(N // NBLK,),
        in_specs=[
            pl.BlockSpec((1, NBLK, D_H), lambda i: (0, i, 0)),
            pl.BlockSpec((1, NBLK, D_H), lambda i: (1, i, 0)),
            pl.BlockSpec((1, NBLK, 16), lambda i: (0, i, 0)),
            pl.BlockSpec((1, NBLK, 16), lambda i: (1, i, 0)),
            pl.BlockSpec((1, D_H), lambda i: (0, 0)),
            pl.BlockSpec((D_H, D_H), lambda i: (0, 0)),
        ],
        out_specs=pl.BlockSpec((NBLK, D_H), lambda i: (i, 0)),
        out_shape=jax.ShapeDtypeStruct((N, D_H), jnp.float32),
    )(num, num, den, den, b, W)


def _final_body(n0_ref, n1_ref, d0_ref, d1_ref, b_ref, wl_ref, bl_ref,
                out_ref):
    h = _combine(n0_ref, n1_ref, d0_ref, d1_ref, b_ref)
    out_ref[...] = jnp.dot(h, wl_ref[...],
                           preferred_element_type=jnp.float32) + bl_ref[...]


def _final_call(num, den, b, Wl, bl):
    return pl.pallas_call(
        _final_body,
        grid=(N // NBLK,),
        in_specs=[
            pl.BlockSpec((1, NBLK, D_H), lambda i: (0, i, 0)),
            pl.BlockSpec((1, NBLK, D_H), lambda i: (1, i, 0)),
            pl.BlockSpec((1, NBLK, 16), lambda i: (0, i, 0)),
            pl.BlockSpec((1, NBLK, 16), lambda i: (1, i, 0)),
            pl.BlockSpec((1, D_H), lambda i: (0, 0)),
            pl.BlockSpec((D_H, D_OUT), lambda i: (0, 0)),
            pl.BlockSpec((1, D_OUT), lambda i: (0, 0)),
        ],
        out_specs=pl.BlockSpec((NBLK, D_OUT), lambda i: (i, 0)),
        out_shape=jax.ShapeDtypeStruct((N, D_OUT), jnp.float32),
    )(num, num, den, den, b, Wl, bl)


def _edge_sc_body(h_hbm, asad_hbm, ei_hbm, ae_hbm, flag_hbm, zero_hbm,
                  num_hbm, den_hbm,
                  as_v, ad_v, src_v, dst_v, ae_v, flag_v, gbuf, sbuf, s_sh,
                  gsem, ssem):
    cid = lax.axis_index("c")
    sid = lax.axis_index("s")
    wid = cid * NS + sid
    row0 = sid * ROWS

    pltpu.sync_copy(flag_hbm, flag_v)
    pltpu.sync_copy(asad_hbm.at[0], as_v)
    pltpu.sync_copy(asad_hbm.at[1], ad_v)
    pltpu.sync_copy(ei_hbm.at[0, wid], src_v)
    pltpu.sync_copy(ei_hbm.at[1, wid], dst_v)
    lidx = flag_v[...][0]
    pltpu.sync_copy(ae_hbm.at[lidx, wid], ae_v)
    pltpu.sync_copy(zero_hbm.at[pl.ds(row0, ROWS)],
                    s_sh.at[pl.ds(row0, ROWS)])
    plsc.subcore_barrier()

    def start_gather(j, b):
        pltpu.async_copy(h_hbm.at[src_v.at[j]], gbuf.at[b],
                         gsem.at[b])

    def wait_gather(j, b):
        pltpu.make_async_copy(h_hbm.at[src_v.at[j]],
                              gbuf.at[b], gsem.at[b]).wait()

    def start_scatter(j, b):
        pltpu.async_copy(sbuf.at[b], s_sh.at[dst_v.at[j]], ssem.at[b],
                         add=True)

    def wait_scatter(j, b):
        pltpu.make_async_copy(sbuf.at[b], s_sh.at[dst_v.at[j]],
                              ssem.at[b]).wait()

    start_gather(0, 0)
    start_gather(1, 1)
    zi = jnp.zeros((L,), jnp.int32)

    def slot(j, b):
        wait_gather(j, b)

        @pl.when(j >= 2)
        def _():
            wait_scatter(j - 2, b)

        for g in range(G):
            si = src_v[j, pl.ds(g * L, L)]
            di = dst_v[j, pl.ds(g * L, L)]
            aev = ae_v[j, pl.ds(g * L, L)]
            t = (plsc.load_gather(as_v, [si])
                 + plsc.load_gather(ad_v, [di]) + aev)
            t = jnp.maximum(t, SLOPE * t)
            w = jnp.exp(t)
            for r in range(L):
                row = g * L + r
                wr = jnp.full((L,), w[r], jnp.float32)
                sbuf[b, row, pl.ds(4 * L, L)] = wr
                for c in range(4):
                    sbuf[b, row, pl.ds(c * L, L)] = (
                        gbuf[b, row, pl.ds(c * L, L)] * wr)

        @pl.when(j + 2 < CH)
        def _():
            start_gather(j + 2, b)

        start_scatter(j, b)

    def loop_body(i, carry):
        j0 = 2 * i
        slot(j0, 0)

        @pl.when(j0 + 1 < CH)
        def _():
            slot(j0 + 1, 1)

        return carry

    lax.fori_loop(0, (CH + 1) // 2, loop_body, 0)
    wait_scatter(CH - 2, (CH - 2) % 2)
    wait_scatter(CH - 1, (CH - 1) % 2)
    plsc.subcore_barrier()
    for i in range(ROWS // K):
        r0 = row0 + i * K
        pltpu.sync_copy(s_sh.at[pl.ds(r0, K)], sbuf.at[0])
        pltpu.sync_copy(sbuf.at[0, :, pl.ds(0, D_H)],
                        num_hbm.at[cid, pl.ds(r0, K)])
        pltpu.sync_copy(sbuf.at[0, :, pl.ds(D_H, 16)],
                        den_hbm.at[cid, pl.ds(r0, K)])


_edge_sc_call = functools.partial(
    pl.kernel,
    out_type=[
        jax.ShapeDtypeStruct((NC, NP, D_H), jnp.float32),
        jax.ShapeDtypeStruct((NC, NP, 16), jnp.float32),
    ],
    mesh=plsc.VectorSubcoreMesh(core_axis_name="c", subcore_axis_name="s"),
    compiler_params=pltpu.CompilerParams(needs_layout_passes=False,
                                         use_tc_tiling_on_sc=False),
    scratch_types=[
        pltpu.VMEM((N,), jnp.float32),
        pltpu.VMEM((N,), jnp.float32),
        pltpu.VMEM((CH, K), jnp.int32),
        pltpu.VMEM((CH, K), jnp.int32),
        pltpu.VMEM((CH, K), jnp.float32),
        pltpu.VMEM((L,), jnp.int32),
        pltpu.VMEM((2, K, D_H), jnp.float32),
        pltpu.VMEM((2, K, SW), jnp.float32),
        pltpu.VMEM_SHARED((NP, SW), jnp.float32),
        pltpu.SemaphoreType.DMA((2,)),
        pltpu.SemaphoreType.DMA((2,)),
    ],
)(_edge_sc_body)


def kernel(x, edge_index, edge_attr, W1, a_src1, a_dst1, We1, ae1, b1,
           W2, a_src2, a_dst2, We2, ae2, b2, Wl, bl):
    f32 = jnp.float32
    Wstack = (jnp.zeros((8, D_EDGE), f32)
              .at[0, :].set(We1 @ ae1)
              .at[1, :].set(We2 @ ae2))

    aev4 = _edgevec_call(edge_attr.T, Wstack).reshape(8, NW, CH, K)
    ei4 = edge_index.reshape(2, NW, CH, K)
    zeros = jnp.zeros((NP, SW), f32)
    flag0 = jnp.zeros((L,), jnp.int32)
    flag1 = jnp.ones((L,), jnp.int32)
    A1 = jnp.zeros((8, D_H), f32).at[0].set(a_src1).at[1].set(a_dst1)
    A2 = jnp.zeros((8, D_H), f32).at[0].set(a_src2).at[1].set(a_dst2)

    h1 = _node_call(x, W1)
    asad1 = _attn_call(h1, A1)
    num1, den1 = _edge_sc_call(h1, asad1, ei4, aev4, flag0, zeros)
    h2 = _mid_call(num1, den1, b1.reshape(1, D_H), W2)
    asad2 = _attn_call(h2, A2)
    num2, den2 = _edge_sc_call(h2, asad2, ei4, aev4, flag1, zeros)
    return _final_call(num2, den2, b2.reshape(1, D_H), Wl,
                       bl.reshape(1, D_OUT))


# speedup vs baseline: 1.2391x; 1.0386x over previous
import functools

import jax
import jax.numpy as jnp
from jax import lax
from jax.experimental import pallas as pl
from jax.experimental.pallas import tpu as pltpu
from jax.experimental.pallas import tpu_sc as plsc

N = 10000
E = 320000
D_IN = 128
D_H = 64
D_OUT = 64
D_EDGE = 4

NC = 2
NS = 16
NW = NC * NS
L = 16
EPT = E // NW
K = 80
CH = EPT // K
G = K // L
NP = 10240
ROWS = NP // NS
SW = 80
NBLK = 5000
EBLK = 320000
EPS = 1e-16
SLOPE = 0.2


def _node_body(x_ref, w_ref, a_ref, h_ref, at_ref):
    h = jnp.dot(x_ref[...], w_ref[...], preferred_element_type=jnp.float32)
    h_ref[...] = h
    at_ref[...] = lax.dot_general(
        a_ref[...], h, (((1,), (1,)), ((), ())),
        preferred_element_type=jnp.float32)


def _node_call(x, W, A8):
    d_in = x.shape[1]
    return pl.pallas_call(
        _node_body,
        in_specs=[
            pl.BlockSpec((N, d_in), lambda: (0, 0)),
            pl.BlockSpec((d_in, D_H), lambda: (0, 0)),
            pl.BlockSpec((8, D_H), lambda: (0, 0)),
        ],
        out_specs=[
            pl.BlockSpec((N, D_H), lambda: (0, 0)),
            pl.BlockSpec((8, N), lambda: (0, 0)),
        ],
        out_shape=[
            jax.ShapeDtypeStruct((N, D_H), jnp.float32),
            jax.ShapeDtypeStruct((8, N), jnp.float32),
        ],
    )(x, W, A8)


def _edgevec_body(ea_ref, ws_ref, out_ref):
    out_ref[...] = jnp.dot(ws_ref[...], ea_ref[...],
                           preferred_element_type=jnp.float32)


def _edgevec_call(eaT, Wstack):
    return pl.pallas_call(
        _edgevec_body,
        grid=(E // EBLK,),
        in_specs=[
            pl.BlockSpec((D_EDGE, EBLK), lambda i: (0, i)),
            pl.BlockSpec((8, D_EDGE), lambda i: (0, 0)),
        ],
        out_specs=pl.BlockSpec((8, EBLK), lambda i: (0, i)),
        out_shape=jax.ShapeDtypeStruct((8, E), jnp.float32),
    )(eaT, Wstack)


def _combine(n0_ref, n1_ref, d0_ref, d1_ref, b_ref):
    num = n0_ref[0] + n1_ref[0]
    den = d0_ref[0][:, 0:1] + d1_ref[0][:, 0:1]
    return num / (den + EPS) + b_ref[...]


def _mid_body(n0_ref, n1_ref, d0_ref, d1_ref, b_ref, w_ref, a_ref,
              h_ref, at_ref):
    h1 = jnp.maximum(_combine(n0_ref, n1_ref, d0_ref, d1_ref, b_ref), 0.0)
    h2 = jnp.dot(h1, w_ref[...], preferred_element_type=jnp.float32)
    h_ref[...] = h2
    at_ref[...] = lax.dot_general(
        a_ref[...], h2, (((1,), (1,)), ((), ())),
        preferred_element_type=jnp.float32)


def _mid_call(num, den, b, W, A8):
    return pl.pallas_call(
        _mid_body,
        grid=(1,),
        in_specs=[
            pl.BlockSpec((1, N, D_H), lambda i: (0, 0, 0)),
            pl.BlockSpec((1, N, D_H), lambda i: (1, 0, 0)),
            pl.BlockSpec((1, N, 16), lambda i: (0, 0, 0)),
            pl.BlockSpec((1, N, 16), lambda i: (1, 0, 0)),
            pl.BlockSpec((1, D_H), lambda i: (0, 0)),
            pl.BlockSpec((D_H, D_H), lambda i: (0, 0)),
            pl.BlockSpec((8, D_H), lambda i: (0, 0)),
        ],
        out_specs=[
            pl.BlockSpec((N, D_H), lambda i: (0, 0)),
            pl.BlockSpec((8, N), lambda i: (0, 0)),
        ],
        out_shape=[
            jax.ShapeDtypeStruct((N, D_H), jnp.float32),
            jax.ShapeDtypeStruct((8, N), jnp.float32),
        ],
    )(num, num, den, den, b, W, A8)


def _final_body(n0_ref, n1_ref, d0_ref, d1_ref, b_ref, wl_ref, bl_ref,
                out_ref):
    h = _combine(n0_ref, n1_ref, d0_ref, d1_ref, b_ref)
    out_ref[...] = jnp.dot(h, wl_ref[...],
                           preferred_element_type=jnp.float32) + bl_ref[...]


def _final_call(num, den, b, Wl, bl):
    return pl.pallas_call(
        _final_body,
        grid=(1,),
        in_specs=[
            pl.BlockSpec((1, N, D_H), lambda i: (0, 0, 0)),
            pl.BlockSpec((1, N, D_H), lambda i: (1, 0, 0)),
            pl.BlockSpec((1, N, 16), lambda i: (0, 0, 0)),
            pl.BlockSpec((1, N, 16), lambda i: (1, 0, 0)),
            pl.BlockSpec((1, D_H), lambda i: (0, 0)),
            pl.BlockSpec((D_H, D_OUT), lambda i: (0, 0)),
            pl.BlockSpec((1, D_OUT), lambda i: (0, 0)),
        ],
        out_specs=pl.BlockSpec((N, D_OUT), lambda i: (0, 0)),
        out_shape=jax.ShapeDtypeStruct((N, D_OUT), jnp.float32),
    )(num, num, den, den, b, Wl, bl)


def _edge_sc_body(h_hbm, asad_hbm, ei_hbm, ae_hbm, flag_hbm, zero_hbm,
                  num_hbm, den_hbm,
                  as_v, ad_v, src_v, dst_v, ae_v, flag_v, gbuf, sbuf, s_sh,
                  gsem, ssem):
    cid = lax.axis_index("c")
    sid = lax.axis_index("s")
    wid = cid * NS + sid
    row0 = sid * ROWS

    pltpu.sync_copy(flag_hbm, flag_v)
    pltpu.sync_copy(asad_hbm.at[0], as_v)
    pltpu.sync_copy(asad_hbm.at[1], ad_v)
    pltpu.sync_copy(ei_hbm.at[0, wid], src_v)
    pltpu.sync_copy(ei_hbm.at[1, wid], dst_v)
    lidx = flag_v[...][0]
    pltpu.sync_copy(ae_hbm.at[lidx, wid], ae_v)
    pltpu.sync_copy(zero_hbm.at[pl.ds(row0, ROWS)],
                    s_sh.at[pl.ds(row0, ROWS)])
    plsc.subcore_barrier()

    def start_gather(j, b):
        pltpu.async_copy(h_hbm.at[src_v.at[j]], gbuf.at[b],
                         gsem.at[b])

    def wait_gather(j, b):
        pltpu.make_async_copy(h_hbm.at[src_v.at[j]],
                              gbuf.at[b], gsem.at[b]).wait()

    def start_scatter(j, b):
        pltpu.async_copy(sbuf.at[b], s_sh.at[dst_v.at[j]], ssem.at[b],
                         add=True)

    def wait_scatter(j, b):
        pltpu.make_async_copy(sbuf.at[b], s_sh.at[dst_v.at[j]],
                              ssem.at[b]).wait()

    start_gather(0, 0)
    start_gather(1, 1)
    zi = jnp.zeros((L,), jnp.int32)

    def slot(j, b):
        wait_gather(j, b)

        @pl.when(j >= 2)
        def _():
            wait_scatter(j - 2, b)

        for g in range(G):
            si = src_v[j, pl.ds(g * L, L)]
            di = dst_v[j, pl.ds(g * L, L)]
            aev = ae_v[j, pl.ds(g * L, L)]
            t = (plsc.load_gather(as_v, [si])
                 + plsc.load_gather(ad_v, [di]) + aev)
            t = jnp.maximum(t, SLOPE * t)
            w = jnp.exp(t)
            for r in range(L):
                row = g * L + r
                wr = jnp.full((L,), w[r], jnp.float32)
                sbuf[b, row, pl.ds(4 * L, L)] = wr
                for c in range(4):
                    sbuf[b, row, pl.ds(c * L, L)] = (
                        gbuf[b, row, pl.ds(c * L, L)] * wr)

        @pl.when(j + 2 < CH)
        def _():
            start_gather(j + 2, b)

        start_scatter(j, b)

    def loop_body(i, carry):
        j0 = 2 * i
        slot(j0, 0)

        @pl.when(j0 + 1 < CH)
        def _():
            slot(j0 + 1, 1)

        return carry

    lax.fori_loop(0, (CH + 1) // 2, loop_body, 0)
    wait_scatter(CH - 2, (CH - 2) % 2)
    wait_scatter(CH - 1, (CH - 1) % 2)
    plsc.subcore_barrier()
    for i in range(ROWS // K):
        r0 = row0 + i * K
        pltpu.sync_copy(s_sh.at[pl.ds(r0, K)], sbuf.at[0])
        pltpu.sync_copy(sbuf.at[0, :, pl.ds(0, D_H)],
                        num_hbm.at[cid, pl.ds(r0, K)])
        pltpu.sync_copy(sbuf.at[0, :, pl.ds(D_H, 16)],
                        den_hbm.at[cid, pl.ds(r0, K)])


_edge_sc_call = functools.partial(
    pl.kernel,
    out_type=[
        jax.ShapeDtypeStruct((NC, NP, D_H), jnp.float32),
        jax.ShapeDtypeStruct((NC, NP, 16), jnp.float32),
    ],
    mesh=plsc.VectorSubcoreMesh(core_axis_name="c", subcore_axis_name="s"),
    compiler_params=pltpu.CompilerParams(needs_layout_passes=False,
                                         use_tc_tiling_on_sc=False),
    scratch_types=[
        pltpu.VMEM((N,), jnp.float32),
        pltpu.VMEM((N,), jnp.float32),
        pltpu.VMEM((CH, K), jnp.int32),
        pltpu.VMEM((CH, K), jnp.int32),
        pltpu.VMEM((CH, K), jnp.float32),
        pltpu.VMEM((L,), jnp.int32),
        pltpu.VMEM((2, K, D_H), jnp.float32),
        pltpu.VMEM((2, K, SW), jnp.float32),
        pltpu.VMEM_SHARED((NP, SW), jnp.float32),
        pltpu.SemaphoreType.DMA((2,)),
        pltpu.SemaphoreType.DMA((2,)),
    ],
)(_edge_sc_body)


def kernel(x, edge_index, edge_attr, W1, a_src1, a_dst1, We1, ae1, b1,
           W2, a_src2, a_dst2, We2, ae2, b2, Wl, bl):
    f32 = jnp.float32
    Wstack = (jnp.zeros((8, D_EDGE), f32)
              .at[0, :].set(We1 @ ae1)
              .at[1, :].set(We2 @ ae2))

    aev4 = _edgevec_call(edge_attr.T, Wstack).reshape(8, NW, CH, K)
    ei4 = edge_index.reshape(2, NW, CH, K)
    zeros = jnp.zeros((NP, SW), f32)
    flag0 = jnp.zeros((L,), jnp.int32)
    flag1 = jnp.ones((L,), jnp.int32)
    A1 = jnp.zeros((8, D_H), f32).at[0].set(a_src1).at[1].set(a_dst1)
    A2 = jnp.zeros((8, D_H), f32).at[0].set(a_src2).at[1].set(a_dst2)

    h1, asad1 = _node_call(x, W1, A1)
    num1, den1 = _edge_sc_call(h1, asad1, ei4, aev4, flag0, zeros)
    h2, asad2 = _mid_call(num1, den1, b1.reshape(1, D_H), W2, A2)
    num2, den2 = _edge_sc_call(h2, asad2, ei4, aev4, flag1, zeros)
    return _final_call(num2, den2, b2.reshape(1, D_H), Wl,
                       bl.reshape(1, D_OUT))
